# baseline (speedup 1.0000x reference)
import jax
import jax.numpy as jnp
from jax.experimental import pallas as pl

_THRESH = 0.8
_ROWS = 1048576
_COLS = 64
_BLOCK_ROWS = 8192


def _body(x_ref, o_ref):
    x = x_ref[...]
    m = jnp.mean(jnp.abs(x), axis=1, keepdims=True)
    o_ref[...] = jnp.where(m > _THRESH, x, 0.0)


def kernel(input_tensor):
    grid = _ROWS // _BLOCK_ROWS
    return pl.pallas_call(
        _body,
        grid=(grid,),
        in_specs=[pl.BlockSpec((_BLOCK_ROWS, _COLS), lambda i: (i, 0))],
        out_specs=pl.BlockSpec((_BLOCK_ROWS, _COLS), lambda i: (i, 0)),
        out_shape=jax.ShapeDtypeStruct((_ROWS, _COLS), jnp.float32),
    )(input_tensor)
